# baseline (speedup 1.0000x reference)
import functools

import jax
import jax.numpy as jnp
from jax import lax
from jax.experimental import pallas as pl
from jax.experimental.pallas import tpu as pltpu
from jax.experimental.pallas import tpu_sc as plsc

VOCAB = 1000000
EMBED_DIM = 128
FF_DIM = 100
OUT_DIM = 9
PAD_DIM = 16
PROWS = VOCAB // 8

NC = 2
NS = 16
NW = NC * NS

PBLK = 1000


def _table_body(e0, e1, e2, e3, e4, e5, e6, e7, w1_ref, b1_ref, w2_ref, b2_ref,
                out_ref):
    w1 = w1_ref[...]
    b1 = b1_ref[...]
    w2 = w2_ref[...]
    b2 = b2_ref[...]
    cols = []
    for e_ref in (e0, e1, e2, e3, e4, e5, e6, e7):
        h = jnp.dot(e_ref[...], w1, preferred_element_type=jnp.float32)
        h = jnp.maximum(h + b1, 0.0)
        cols.append(jnp.dot(h, w2, preferred_element_type=jnp.float32) + b2)
    out_ref[...] = jnp.concatenate(cols, axis=1)


def _make_table(emb, w1, b1, w2p, b2p):
    grid = PROWS // PBLK
    emb_specs = [
        pl.BlockSpec((PBLK, EMBED_DIM), functools.partial(lambda q, i: (q * grid + i, 0), q))
        for q in range(8)
    ]
    return pl.pallas_call(
        _table_body,
        grid=(grid,),
        in_specs=emb_specs + [
            pl.BlockSpec((EMBED_DIM, FF_DIM), lambda i: (0, 0)),
            pl.BlockSpec((1, FF_DIM), lambda i: (0, 0)),
            pl.BlockSpec((FF_DIM, PAD_DIM), lambda i: (0, 0)),
            pl.BlockSpec((1, PAD_DIM), lambda i: (0, 0)),
        ],
        out_specs=pl.BlockSpec((PBLK, 128), lambda i: (i, 0)),
        out_shape=jax.ShapeDtypeStruct((PROWS, 128), jnp.float32),
    )(*([emb] * 8), w1, b1, w2p, b2p)


def _idx_body(x_ref, out_ref):
    v = x_ref[...]
    q = lax.div(v, PROWS)
    p = v - q * PROWS
    out_ref[...] = p * 8 + q


def _transform_idx(xf, rows, cols):
    x2 = xf.reshape(rows, cols)
    return pl.pallas_call(
        _idx_body,
        grid=(1,),
        in_specs=[pl.BlockSpec((rows, cols), lambda i: (0, 0))],
        out_specs=pl.BlockSpec((rows, cols), lambda i: (0, 0)),
        out_shape=jax.ShapeDtypeStruct((rows, cols), jnp.int32),
    )(x2).reshape(-1)


def _make_gather(n_tokens, chunk, nbuf):
    b_per_w = n_tokens // NW
    n_chunks = b_per_w // chunk
    mesh = plsc.VectorSubcoreMesh(core_axis_name="c", subcore_axis_name="s")

    @functools.partial(
        pl.kernel,
        out_type=jax.ShapeDtypeStruct((n_tokens, PAD_DIM), jnp.float32),
        mesh=mesh,
        compiler_params=pltpu.CompilerParams(use_tc_tiling_on_sc=False),
        scratch_types=(
            [pltpu.VMEM((b_per_w,), jnp.int32)]
            + [pltpu.VMEM((chunk, PAD_DIM), jnp.float32)] * nbuf
            + [pltpu.SemaphoreType.DMA] * (2 * nbuf)
        ),
    )
    def gather_kernel(table_hbm, idx_hbm, out_hbm, idx_all, *bufs):
        rows = bufs[:nbuf]
        gsem = bufs[nbuf:2 * nbuf]
        osem = bufs[2 * nbuf:3 * nbuf]
        wid = lax.axis_index("s") * NC + lax.axis_index("c")
        base = wid * b_per_w
        pltpu.sync_copy(idx_hbm.at[pl.ds(base, b_per_w)], idx_all)

        gh = [None] * nbuf
        oh = [None] * nbuf
        for j in range(min(nbuf - 1, n_chunks)):
            gh[j] = pltpu.async_copy(
                table_hbm.at[idx_all.at[pl.ds(j * chunk, chunk)]],
                rows[j], gsem[j])
        for i in range(n_chunks):
            b = i % nbuf
            nxt = i + nbuf - 1
            if nxt < n_chunks:
                nb = nxt % nbuf
                if oh[nb] is not None:
                    oh[nb].wait()
                    oh[nb] = None
                gh[nb] = pltpu.async_copy(
                    table_hbm.at[idx_all.at[pl.ds(nxt * chunk, chunk)]],
                    rows[nb], gsem[nb])
            gh[b].wait()
            oh[b] = pltpu.async_copy(
                rows[b],
                out_hbm.at[pl.ds(base + i * chunk, chunk)], osem[b])
        for j in range(nbuf):
            if oh[j] is not None:
                oh[j].wait()

    return gather_kernel


def kernel(x, emb, W1, b1, W2, b2):
    B, S = x.shape
    w2p = jnp.pad(W2, ((0, 0), (0, PAD_DIM - OUT_DIM)))
    b2p = jnp.pad(b2, (0, PAD_DIM - OUT_DIM)).reshape(1, PAD_DIM)
    b1r = b1.reshape(1, FF_DIM)

    packed = _make_table(emb, W1, b1r, w2p, b2p)
    table = packed.reshape(VOCAB, PAD_DIM)

    xf = x.reshape(-1).astype(jnp.int32)
    xf2 = _transform_idx(xf, 100, (B * S) // 100)
    gathered = _make_gather(xf.shape[0], 1280, 4)(table, xf2)
    return gathered[:, :OUT_DIM].reshape(B, S, OUT_DIM)


# speedup vs baseline: 1.0002x; 1.0002x over previous
import functools

import jax
import jax.numpy as jnp
from jax import lax
from jax.experimental import pallas as pl
from jax.experimental.pallas import tpu as pltpu
from jax.experimental.pallas import tpu_sc as plsc

VOCAB = 1000000
EMBED_DIM = 128
FF_DIM = 100
OUT_DIM = 9
PAD_DIM = 16
PROWS = VOCAB // 8

NC = 2
NS = 16
NW = NC * NS

PBLK = 1000


def _table_body(e0, e1, e2, e3, e4, e5, e6, e7, w1_ref, b1_ref, w2_ref, b2_ref,
                out_ref):
    w1 = w1_ref[...]
    b1 = b1_ref[...]
    w2 = w2_ref[...]
    b2 = b2_ref[...]
    cols = []
    for e_ref in (e0, e1, e2, e3, e4, e5, e6, e7):
        h = jnp.dot(e_ref[...], w1, preferred_element_type=jnp.float32)
        h = jnp.maximum(h + b1, 0.0)
        cols.append(jnp.dot(h, w2, preferred_element_type=jnp.float32) + b2)
    out_ref[...] = jnp.concatenate(cols, axis=1)


def _make_table(emb, w1, b1, w2p, b2p):
    grid = PROWS // PBLK
    emb_specs = [
        pl.BlockSpec((PBLK, EMBED_DIM), functools.partial(lambda q, i: (q * grid + i, 0), q))
        for q in range(8)
    ]
    return pl.pallas_call(
        _table_body,
        grid=(grid,),
        in_specs=emb_specs + [
            pl.BlockSpec((EMBED_DIM, FF_DIM), lambda i: (0, 0)),
            pl.BlockSpec((1, FF_DIM), lambda i: (0, 0)),
            pl.BlockSpec((FF_DIM, PAD_DIM), lambda i: (0, 0)),
            pl.BlockSpec((1, PAD_DIM), lambda i: (0, 0)),
        ],
        out_specs=pl.BlockSpec((PBLK, 128), lambda i: (i, 0)),
        out_shape=jax.ShapeDtypeStruct((PROWS, 128), jnp.float32),
    )(*([emb] * 8), w1, b1, w2p, b2p)


def _idx_body(x_ref, out_ref):
    v = x_ref[...]
    q = lax.div(v, PROWS)
    p = v - q * PROWS
    out_ref[...] = p * 8 + q


def _transform_idx(xf, rows, cols):
    x2 = xf.reshape(rows, cols)
    return pl.pallas_call(
        _idx_body,
        grid=(1,),
        in_specs=[pl.BlockSpec((rows, cols), lambda i: (0, 0))],
        out_specs=pl.BlockSpec((rows, cols), lambda i: (0, 0)),
        out_shape=jax.ShapeDtypeStruct((rows, cols), jnp.int32),
    )(x2).reshape(-1)


def _make_gather(n_tokens, chunk, nbuf):
    b_per_w = n_tokens // NW
    n_chunks = b_per_w // chunk
    mesh = plsc.VectorSubcoreMesh(core_axis_name="c", subcore_axis_name="s")

    @functools.partial(
        pl.kernel,
        out_type=jax.ShapeDtypeStruct((n_tokens, PAD_DIM), jnp.float32),
        mesh=mesh,
        compiler_params=pltpu.CompilerParams(use_tc_tiling_on_sc=False),
        scratch_types=(
            [pltpu.VMEM((b_per_w,), jnp.int32)]
            + [pltpu.VMEM((chunk, PAD_DIM), jnp.float32)] * nbuf
            + [pltpu.SemaphoreType.DMA] * (2 * nbuf)
        ),
    )
    def gather_kernel(table_hbm, idx_hbm, out_hbm, idx_all, *bufs):
        rows = bufs[:nbuf]
        gsem = bufs[nbuf:2 * nbuf]
        osem = bufs[2 * nbuf:3 * nbuf]
        wid = lax.axis_index("s") * NC + lax.axis_index("c")
        base = wid * b_per_w
        pltpu.sync_copy(idx_hbm.at[pl.ds(base, b_per_w)], idx_all)

        gh = [None] * nbuf
        oh = [None] * nbuf
        for j in range(min(nbuf - 1, n_chunks)):
            gh[j] = pltpu.async_copy(
                table_hbm.at[idx_all.at[pl.ds(j * chunk, chunk)]],
                rows[j], gsem[j])
        for i in range(n_chunks):
            b = i % nbuf
            nxt = i + nbuf - 1
            if nxt < n_chunks:
                nb = nxt % nbuf
                if oh[nb] is not None:
                    oh[nb].wait()
                    oh[nb] = None
                gh[nb] = pltpu.async_copy(
                    table_hbm.at[idx_all.at[pl.ds(nxt * chunk, chunk)]],
                    rows[nb], gsem[nb])
            gh[b].wait()
            oh[b] = pltpu.async_copy(
                rows[b],
                out_hbm.at[pl.ds(base + i * chunk, chunk)], osem[b])
        for j in range(nbuf):
            if oh[j] is not None:
                oh[j].wait()

    return gather_kernel


def kernel(x, emb, W1, b1, W2, b2):
    B, S = x.shape
    w2p = jnp.pad(W2, ((0, 0), (0, PAD_DIM - OUT_DIM)))
    b2p = jnp.pad(b2, (0, PAD_DIM - OUT_DIM)).reshape(1, PAD_DIM)
    b1r = b1.reshape(1, FF_DIM)

    packed = _make_table(emb, W1, b1r, w2p, b2p)
    table = packed.reshape(VOCAB, PAD_DIM)

    xf = x.reshape(-1).astype(jnp.int32)
    xf2 = _transform_idx(xf, 100, (B * S) // 100)
    gathered = _make_gather(xf.shape[0], 1280, 5)(table, xf2)
    return gathered[:, :OUT_DIM].reshape(B, S, OUT_DIM)
